# baseline (speedup 1.0000x reference)
import functools

import jax
import jax.numpy as jnp
from jax import lax
from jax.experimental import pallas as pl
from jax.experimental.pallas import tpu as pltpu
from jax.experimental.pallas import tpu_sc as plsc

_N = 100000
_T = 95
_L = 16
_NC = 2
_NS = 16
_NW = _NC * _NS
_CHUNK = 6272
_HALF = _CHUNK // 2

_mesh = plsc.VectorSubcoreMesh(core_axis_name="c", subcore_axis_name="s", num_cores=1)


@functools.partial(
    pl.kernel,
    mesh=_mesh,
    out_type=jax.ShapeDtypeStruct((_N,), jnp.float32),
    compiler_params=pltpu.CompilerParams(needs_layout_passes=False),
    scratch_types=[
        pltpu.VMEM((_T,), jnp.float32),
        pltpu.VMEM((_CHUNK,), jnp.int32),
        pltpu.VMEM((_CHUNK,), jnp.float32),
        pltpu.SemaphoreType.DMA,
        pltpu.SemaphoreType.DMA,
        pltpu.SemaphoreType.DMA,
        pltpu.SemaphoreType.DMA,
    ],
)
def _atomref_sc(types_hbm, table_hbm, out_hbm, table_v, idx_v, out_v,
                sem_t, sem_i0, sem_i1, sem_o):
    wid = lax.axis_index("s")
    base = lax.min(wid * _CHUNK, _N - _CHUNK)

    ct = pltpu.async_copy(table_hbm, table_v, sem_t)
    ci0 = pltpu.async_copy(
        types_hbm.at[pl.ds(base, _HALF)], idx_v.at[pl.ds(0, _HALF)], sem_i0)
    ci1 = pltpu.async_copy(
        types_hbm.at[pl.ds(base + _HALF, _HALF)],
        idx_v.at[pl.ds(_HALF, _HALF)], sem_i1)
    ct.wait()
    ci0.wait()

    @plsc.parallel_loop(0, _HALF, _L, unroll=8)
    def _body0(i):
        idx = idx_v[pl.ds(i, _L)]
        out_v[pl.ds(i, _L)] = plsc.load_gather(table_v, [idx])

    co0 = pltpu.async_copy(
        out_v.at[pl.ds(0, _HALF)], out_hbm.at[pl.ds(base, _HALF)], sem_o)
    ci1.wait()

    @plsc.parallel_loop(_HALF, _CHUNK, _L, unroll=8)
    def _body1(i):
        idx = idx_v[pl.ds(i, _L)]
        out_v[pl.ds(i, _L)] = plsc.load_gather(table_v, [idx])

    co1 = pltpu.async_copy(
        out_v.at[pl.ds(_HALF, _HALF)],
        out_hbm.at[pl.ds(base + _HALF, _HALF)], sem_o)
    co0.wait()
    co1.wait()


def kernel(atom_types, elemental_energies):
    return _atomref_sc(atom_types.astype(jnp.int32), elemental_energies)


# speedup vs baseline: 1.0058x; 1.0058x over previous
import functools

import jax
import jax.numpy as jnp
from jax import lax
from jax.experimental import pallas as pl
from jax.experimental.pallas import tpu as pltpu
from jax.experimental.pallas import tpu_sc as plsc

_N = 100000
_T = 95
_L = 16
_NS = 16
_CHUNK = 6272

_mesh = plsc.VectorSubcoreMesh(
    core_axis_name="c", subcore_axis_name="s", num_cores=1)


@functools.partial(
    pl.kernel,
    mesh=_mesh,
    out_type=jax.ShapeDtypeStruct((_N,), jnp.float32),
    compiler_params=pltpu.CompilerParams(needs_layout_passes=False),
    scratch_types=[
        pltpu.VMEM((_T,), jnp.float32),
        pltpu.VMEM((_CHUNK,), jnp.int32),
        pltpu.VMEM((_CHUNK,), jnp.float32),
        pltpu.SemaphoreType.DMA,
        pltpu.SemaphoreType.DMA,
    ],
)
def _atomref_sc(types_hbm, table_hbm, out_hbm, table_v, idx_v, out_v,
                sem_t, sem_i):
    wid = lax.axis_index("s")
    base = lax.min(wid * _CHUNK, _N - _CHUNK)

    ct = pltpu.async_copy(table_hbm, table_v, sem_t)
    ci = pltpu.async_copy(types_hbm.at[pl.ds(base, _CHUNK)], idx_v, sem_i)
    ct.wait()
    ci.wait()

    @plsc.parallel_loop(0, _CHUNK, _L, unroll=8)
    def _body(i):
        idx = idx_v[pl.ds(i, _L)]
        out_v[pl.ds(i, _L)] = plsc.load_gather(table_v, [idx])

    pltpu.sync_copy(out_v, out_hbm.at[pl.ds(base, _CHUNK)])


def kernel(atom_types, elemental_energies):
    return _atomref_sc(atom_types.astype(jnp.int32), elemental_energies)
